# baseline (speedup 1.0000x reference)
import jax
import jax.numpy as jnp
from jax.experimental import pallas as pl
from jax.experimental.pallas import tpu as pltpu
from jax.experimental.pallas import tpu_sc as plsc

T = 4096
D = 2048
F = 1408
E = 8
SF = 2 * F

BT = 256
NT = 2 * T // BT + E
PAD_P = NT * BT

BF = 704
NF = F // BF
BD = 512
ND = D // BD

RT = 512
BT2 = 512
BSF = 704
NSF = SF // BSF
BD2 = 1024

_f32 = jnp.float32
_bf16 = jnp.bfloat16


def _dot_t(a, b):
    return jax.lax.dot_general(
        a, b, (((1,), (1,)), ((), ())), preferred_element_type=_f32)


def _silu(v):
    return v * jax.nn.sigmoid(v)


def _router_body(x_ref, wg_ref, eid_ref, tw_ref):
    x = x_ref[...]
    logits = jax.lax.dot_general(
        x, wg_ref[...], (((1,), (1,)), ((), ())),
        preferred_element_type=_f32)
    m = jnp.max(logits, axis=1, keepdims=True)
    p = jnp.exp(logits - m)
    s = p / jnp.sum(p, axis=1, keepdims=True)
    iota = jax.lax.broadcasted_iota(jnp.int32, s.shape, 1)
    m1 = jnp.max(s, axis=1, keepdims=True)
    i1 = jnp.min(jnp.where(s == m1, iota, E), axis=1, keepdims=True)
    s2 = jnp.where(iota == i1, -jnp.inf, s)
    m2 = jnp.max(s2, axis=1, keepdims=True)
    i2 = jnp.min(jnp.where(s2 == m2, iota, E), axis=1, keepdims=True)
    denom = m1 + m2
    w1 = m1 / denom
    w2 = m2 / denom
    n = x.shape[0]
    eid_ref[...] = jnp.zeros((8, n), jnp.int32)
    eid_ref[0:1, :] = i1.reshape(1, n)
    eid_ref[1:2, :] = i2.reshape(1, n)
    tw_ref[...] = jnp.zeros((8, n), _f32)
    tw_ref[0:1, :] = w1.reshape(1, n)
    tw_ref[1:2, :] = w2.reshape(1, n)


def _router(x, w_gate):
    return pl.pallas_call(
        _router_body,
        grid=(T // RT,),
        in_specs=[
            pl.BlockSpec((RT, D), lambda t: (t, 0)),
            pl.BlockSpec((E, D), lambda t: (0, 0)),
        ],
        out_specs=[
            pl.BlockSpec((8, RT), lambda t: (0, t)),
            pl.BlockSpec((8, RT), lambda t: (0, t)),
        ],
        out_shape=[
            jax.ShapeDtypeStruct((8, T), jnp.int32),
            jax.ShapeDtypeStruct((8, T), _f32),
        ],
    )(x, w_gate)


def _cumsum_rows_incl(o):
    nb = T // 128
    rowi = jax.lax.broadcasted_iota(jnp.int32, (128, 128), 0)
    coli = jax.lax.broadcasted_iota(jnp.int32, (128, 128), 1)
    tri = (rowi >= coli).astype(_bf16)
    blocks = []
    offs = jnp.zeros((1, E), _f32)
    for b in range(nb):
        ob = o[b * 128:(b + 1) * 128]
        cb = jax.lax.dot_general(
            tri, ob.astype(_bf16), (((1,), (0,)), ((), ())),
            preferred_element_type=_f32)
        blocks.append(cb + offs)
        offs = offs + jnp.sum(ob, axis=0, keepdims=True)
    return jnp.concatenate(blocks, axis=0)


def _cumsum_lanes_incl(v):
    rowi = jax.lax.broadcasted_iota(jnp.int32, (E, E), 0)
    coli = jax.lax.broadcasted_iota(jnp.int32, (E, E), 1)
    tri = (rowi <= coli).astype(_bf16)
    return jax.lax.dot_general(
        v.astype(_bf16), tri, (((1,), (0,)), ((), ())),
        preferred_element_type=_f32)


def _dispatch_body(eid_ref, pos_ref, te_ref):
    e0 = eid_ref[0:1, :].reshape(T, 1)
    e1 = eid_ref[1:2, :].reshape(T, 1)
    iota = jax.lax.broadcasted_iota(jnp.int32, (T, E), 1)
    o0 = (e0 == iota).astype(_f32)
    o1 = (e1 == iota).astype(_f32)
    c0 = _cumsum_rows_incl(o0)
    c1 = _cumsum_rows_incl(o1)
    cnt0 = c0[T - 1:T, :]
    counts = cnt0 + c1[T - 1:T, :]
    r0 = jnp.sum((c0 - o0) * o0, axis=1, keepdims=True)
    r1 = (jnp.sum((c1 - o1) * o1, axis=1, keepdims=True)
          + jnp.sum(o1 * cnt0, axis=1, keepdims=True))
    ntiles_e = jnp.ceil(counts / BT)
    ao_end = _cumsum_lanes_incl(ntiles_e)
    ao_rows = (ao_end - ntiles_e) * BT
    base0 = jnp.sum(o0 * ao_rows, axis=1, keepdims=True)
    base1 = jnp.sum(o1 * ao_rows, axis=1, keepdims=True)
    pos_ref[...] = jnp.zeros((8, T), jnp.int32)
    pos_ref[0:1, :] = (base0 + r0).astype(jnp.int32).reshape(1, T)
    pos_ref[1:2, :] = (base1 + r1).astype(jnp.int32).reshape(1, T)
    ti = jax.lax.broadcasted_iota(jnp.int32, (NT, E), 0).astype(_f32)
    te = jnp.sum((jnp.broadcast_to(ao_end, (NT, E)) <= ti).astype(jnp.int32),
                 axis=1, keepdims=True)
    te_ref[...] = te.reshape(1, NT)


def _dispatch(eid):
    return pl.pallas_call(
        _dispatch_body,
        grid=(1,),
        in_specs=[pl.BlockSpec((8, T), lambda i: (0, 0))],
        out_specs=[
            pl.BlockSpec((8, T), lambda i: (0, 0)),
            pl.BlockSpec((1, NT), lambda i: (0, 0)),
        ],
        out_shape=[
            jax.ShapeDtypeStruct((8, T), jnp.int32),
            jax.ShapeDtypeStruct((1, NT), jnp.int32),
        ],
    )(eid)


_SC_CH = 32
_SC_NSUB = 16


def _vector_mesh():
    return plsc.VectorSubcoreMesh(
        core_axis_name="core", subcore_axis_name="subcore")


def _sc_scatter(xb, posflat):
    stripe = T // _SC_NSUB

    @pl.kernel(
        out_type=jax.ShapeDtypeStruct((PAD_P, D), _f32),
        mesh=_vector_mesh(),
        scratch_types=[
            pltpu.VMEM((_SC_CH,), jnp.int32),
            pltpu.VMEM((_SC_CH, D), _f32),
            pltpu.SemaphoreType.DMA,
        ])
    def run(x_hbm, i_hbm, o_hbm, idx_v, rows_v, sem):
        k = jax.lax.axis_index("core")
        s = jax.lax.axis_index("subcore")
        tok_base = s * stripe

        @pl.loop(0, stripe, step=_SC_CH)
        def _(ci):
            tok0 = tok_base + ci
            pltpu.sync_copy(i_hbm.at[pl.ds(k * T + tok0, _SC_CH)], idx_v)
            pltpu.sync_copy(x_hbm.at[pl.ds(tok0, _SC_CH)], rows_v)
            pltpu.async_copy(rows_v, o_hbm.at[idx_v], sem).wait()

    return run(xb, posflat)


def _sc_gather(y, posflat):
    stripe = T // _SC_NSUB

    @pl.kernel(
        out_type=jax.ShapeDtypeStruct((2 * T, D), _f32),
        mesh=_vector_mesh(),
        scratch_types=[
            pltpu.VMEM((_SC_CH,), jnp.int32),
            pltpu.VMEM((_SC_CH, D), _f32),
            pltpu.SemaphoreType.DMA,
        ])
    def run(y_hbm, i_hbm, o_hbm, idx_v, rows_v, sem):
        k = jax.lax.axis_index("core")
        s = jax.lax.axis_index("subcore")
        row_base = k * T + s * stripe

        @pl.loop(0, stripe, step=_SC_CH)
        def _(ci):
            r0 = row_base + ci
            pltpu.sync_copy(i_hbm.at[pl.ds(r0, _SC_CH)], idx_v)
            pltpu.async_copy(y_hbm.at[idx_v], rows_v, sem).wait()
            pltpu.sync_copy(rows_v, o_hbm.at[pl.ds(r0, _SC_CH)])

    return run(y, posflat)


def _gateup_body(te_ref, xg_ref, wg_ref, wu_ref, h_ref, wgb_ref, wub_ref):
    t = pl.program_id(1)
    te = te_ref[t]
    changed = (t == 0) | (te != te_ref[jnp.maximum(t - 1, 0)])

    @pl.when(changed)
    def _():
        wgb_ref[...] = wg_ref[0].astype(_bf16)
        wub_ref[...] = wu_ref[0].astype(_bf16)

    @pl.when(te < E)
    def _():
        xb = xg_ref[...].astype(_bf16)
        g = _dot_t(xb, wgb_ref[...])
        u = _dot_t(xb, wub_ref[...])
        h_ref[0] = (_silu(g) * u).astype(_bf16)


def _gateup(te, xg, w_gate_up):
    grid_spec = pltpu.PrefetchScalarGridSpec(
        num_scalar_prefetch=1,
        grid=(NF, NT),
        in_specs=[
            pl.BlockSpec((BT, D), lambda f, t, te: (t, 0)),
            pl.BlockSpec((1, BF, D),
                         lambda f, t, te: (jnp.minimum(te[t], E - 1), f, 0)),
            pl.BlockSpec((1, BF, D),
                         lambda f, t, te: (jnp.minimum(te[t], E - 1),
                                           NF + f, 0)),
        ],
        out_specs=pl.BlockSpec((1, BT, BF), lambda f, t, te: (f, t, 0)),
        scratch_shapes=[pltpu.VMEM((BF, D), _bf16),
                        pltpu.VMEM((BF, D), _bf16)],
    )
    return pl.pallas_call(
        _gateup_body,
        grid_spec=grid_spec,
        out_shape=jax.ShapeDtypeStruct((NF, PAD_P, BF), _bf16),
    )(te, xg, w_gate_up, w_gate_up)


def _down_body(te_ref, h_ref, wd_ref, y_ref, wdb_ref):
    t = pl.program_id(0)
    te = te_ref[t]
    changed = (t == 0) | (te != te_ref[jnp.maximum(t - 1, 0)])

    @pl.when(changed)
    def _():
        wdb_ref[...] = wd_ref[0].astype(_bf16)

    @pl.when(te < E)
    def _():
        acc = _dot_t(h_ref[0], wdb_ref[:, 0:BF])
        for c in range(1, NF):
            acc = acc + _dot_t(h_ref[c], wdb_ref[:, c * BF:(c + 1) * BF])
        y_ref[...] = acc


def _down(te, h, w_down):
    grid_spec = pltpu.PrefetchScalarGridSpec(
        num_scalar_prefetch=1,
        grid=(NT,),
        in_specs=[
            pl.BlockSpec((NF, BT, BF), lambda t, te: (0, t, 0)),
            pl.BlockSpec((1, D, F),
                         lambda t, te: (jnp.minimum(te[t], E - 1), 0, 0)),
        ],
        out_specs=pl.BlockSpec((BT, D), lambda t, te: (t, 0)),
        scratch_shapes=[pltpu.VMEM((D, F), _bf16)],
    )
    return pl.pallas_call(
        _down_body,
        grid_spec=grid_spec,
        out_shape=jax.ShapeDtypeStruct((PAD_P, D), _f32),
    )(te, h, w_down)


def _shared_gateup_body(xb_ref, wg_ref, wu_ref, sh_ref, wgb_ref, wub_ref):
    t = pl.program_id(1)

    @pl.when(t == 0)
    def _():
        wgb_ref[...] = wg_ref[...].astype(_bf16)
        wub_ref[...] = wu_ref[...].astype(_bf16)

    xb = xb_ref[...]
    g = _dot_t(xb, wgb_ref[...])
    u = _dot_t(xb, wub_ref[...])
    sh_ref[0] = (_silu(g) * u).astype(_bf16)


def _shared_gateup(xb, w_shared_gate_up):
    return pl.pallas_call(
        _shared_gateup_body,
        grid=(NSF, T // BT2),
        in_specs=[
            pl.BlockSpec((BT2, D), lambda s, t: (t, 0)),
            pl.BlockSpec((BSF, D), lambda s, t: (s, 0)),
            pl.BlockSpec((BSF, D), lambda s, t: (NSF + s, 0)),
        ],
        out_specs=pl.BlockSpec((1, BT2, BSF), lambda s, t: (s, t, 0)),
        out_shape=jax.ShapeDtypeStruct((NSF, T, BSF), _bf16),
        scratch_shapes=[pltpu.VMEM((BSF, D), _bf16),
                        pltpu.VMEM((BSF, D), _bf16)],
    )(xb, w_shared_gate_up, w_shared_gate_up)


def _shared_down_body(sh_ref, wd_ref, y0_ref, y1_ref, tw_ref, out_ref,
                      wdb_ref):
    t = pl.program_id(1)

    @pl.when(t == 0)
    def _():
        wdb_ref[...] = wd_ref[...].astype(_bf16)

    acc = _dot_t(sh_ref[0], wdb_ref[:, 0:BSF])
    for c in range(1, NSF):
        acc = acc + _dot_t(sh_ref[c], wdb_ref[:, c * BSF:(c + 1) * BSF])
    tw0 = tw_ref[0:1, :].reshape(BT2, 1)
    tw1 = tw_ref[1:2, :].reshape(BT2, 1)
    out_ref[...] = acc + tw0 * y0_ref[...] + tw1 * y1_ref[...]


def _shared_down_combine(sh, w_shared_down, yg, tw):
    nt2 = T // BT2
    return pl.pallas_call(
        _shared_down_body,
        grid=(D // BD2, nt2),
        in_specs=[
            pl.BlockSpec((NSF, BT2, BSF), lambda d, t: (0, t, 0)),
            pl.BlockSpec((BD2, SF), lambda d, t: (d, 0),
                         pipeline_mode=pl.Buffered(buffer_count=1)),
            pl.BlockSpec((BT2, BD2), lambda d, t: (t, d)),
            pl.BlockSpec((BT2, BD2), lambda d, t: (nt2 + t, d)),
            pl.BlockSpec((8, BT2), lambda d, t: (0, t)),
        ],
        out_specs=pl.BlockSpec((BT2, BD2), lambda d, t: (t, d)),
        out_shape=jax.ShapeDtypeStruct((T, D), _f32),
        scratch_shapes=[pltpu.VMEM((BD2, SF), _bf16)],
    )(sh, w_shared_down, yg, yg, tw)


def kernel(x, w_gate, w_gate_up, w_down, w_shared_gate_up, w_shared_down):
    xb = x.astype(_bf16)
    eid, tw = _router(x, w_gate)
    pos8, te2d = _dispatch(eid)
    posflat = pos8[:2].reshape(2 * T)
    te = te2d.reshape(NT)
    xg = _sc_scatter(x, posflat)
    h = _gateup(te, xg, w_gate_up)
    y = _down(te, h, w_down)
    yg = _sc_gather(y, posflat)
    sh = _shared_gateup(xb, w_shared_gate_up)
    return _shared_down_combine(sh, w_shared_down, yg, tw)


# speedup vs baseline: 1.0114x; 1.0114x over previous
import jax
import jax.numpy as jnp
from jax.experimental import pallas as pl
from jax.experimental.pallas import tpu as pltpu
from jax.experimental.pallas import tpu_sc as plsc

T = 4096
D = 2048
F = 1408
E = 8
SF = 2 * F

BT = 512
NT = 2 * T // BT + E
PAD_P = NT * BT

BF = 704
NF = F // BF
BD = 512
ND = D // BD

RT = 512
BT2 = 512
BSF = 704
NSF = SF // BSF
BD2 = 1024

_f32 = jnp.float32
_bf16 = jnp.bfloat16


def _dot_t(a, b):
    return jax.lax.dot_general(
        a, b, (((1,), (1,)), ((), ())), preferred_element_type=_f32)


def _silu(v):
    return v * jax.nn.sigmoid(v)


def _router_body(x_ref, wg_ref, eid_ref, tw_ref):
    x = x_ref[...]
    logits = jax.lax.dot_general(
        x, wg_ref[...], (((1,), (1,)), ((), ())),
        preferred_element_type=_f32)
    m = jnp.max(logits, axis=1, keepdims=True)
    p = jnp.exp(logits - m)
    s = p / jnp.sum(p, axis=1, keepdims=True)
    iota = jax.lax.broadcasted_iota(jnp.int32, s.shape, 1)
    m1 = jnp.max(s, axis=1, keepdims=True)
    i1 = jnp.min(jnp.where(s == m1, iota, E), axis=1, keepdims=True)
    s2 = jnp.where(iota == i1, -jnp.inf, s)
    m2 = jnp.max(s2, axis=1, keepdims=True)
    i2 = jnp.min(jnp.where(s2 == m2, iota, E), axis=1, keepdims=True)
    denom = m1 + m2
    w1 = m1 / denom
    w2 = m2 / denom
    n = x.shape[0]
    eid_ref[...] = jnp.zeros((8, n), jnp.int32)
    eid_ref[0:1, :] = i1.reshape(1, n)
    eid_ref[1:2, :] = i2.reshape(1, n)
    tw_ref[...] = jnp.zeros((8, n), _f32)
    tw_ref[0:1, :] = w1.reshape(1, n)
    tw_ref[1:2, :] = w2.reshape(1, n)


def _router(x, w_gate):
    return pl.pallas_call(
        _router_body,
        grid=(T // RT,),
        in_specs=[
            pl.BlockSpec((RT, D), lambda t: (t, 0)),
            pl.BlockSpec((E, D), lambda t: (0, 0)),
        ],
        out_specs=[
            pl.BlockSpec((8, RT), lambda t: (0, t)),
            pl.BlockSpec((8, RT), lambda t: (0, t)),
        ],
        out_shape=[
            jax.ShapeDtypeStruct((8, T), jnp.int32),
            jax.ShapeDtypeStruct((8, T), _f32),
        ],
    )(x, w_gate)


def _cumsum_rows_incl(o):
    nb = T // 128
    rowi = jax.lax.broadcasted_iota(jnp.int32, (128, 128), 0)
    coli = jax.lax.broadcasted_iota(jnp.int32, (128, 128), 1)
    tri = (rowi >= coli).astype(_bf16)
    blocks = []
    offs = jnp.zeros((1, E), _f32)
    for b in range(nb):
        ob = o[b * 128:(b + 1) * 128]
        cb = jax.lax.dot_general(
            tri, ob.astype(_bf16), (((1,), (0,)), ((), ())),
            preferred_element_type=_f32)
        blocks.append(cb + offs)
        offs = offs + jnp.sum(ob, axis=0, keepdims=True)
    return jnp.concatenate(blocks, axis=0)


def _cumsum_lanes_incl(v):
    rowi = jax.lax.broadcasted_iota(jnp.int32, (E, E), 0)
    coli = jax.lax.broadcasted_iota(jnp.int32, (E, E), 1)
    tri = (rowi <= coli).astype(_bf16)
    return jax.lax.dot_general(
        v.astype(_bf16), tri, (((1,), (0,)), ((), ())),
        preferred_element_type=_f32)


def _dispatch_body(eid_ref, pos_ref, te_ref):
    e0 = eid_ref[0:1, :].reshape(T, 1)
    e1 = eid_ref[1:2, :].reshape(T, 1)
    iota = jax.lax.broadcasted_iota(jnp.int32, (T, E), 1)
    o0 = (e0 == iota).astype(_f32)
    o1 = (e1 == iota).astype(_f32)
    c0 = _cumsum_rows_incl(o0)
    c1 = _cumsum_rows_incl(o1)
    cnt0 = c0[T - 1:T, :]
    counts = cnt0 + c1[T - 1:T, :]
    r0 = jnp.sum((c0 - o0) * o0, axis=1, keepdims=True)
    r1 = (jnp.sum((c1 - o1) * o1, axis=1, keepdims=True)
          + jnp.sum(o1 * cnt0, axis=1, keepdims=True))
    ntiles_e = jnp.ceil(counts / BT)
    ao_end = _cumsum_lanes_incl(ntiles_e)
    ao_rows = (ao_end - ntiles_e) * BT
    base0 = jnp.sum(o0 * ao_rows, axis=1, keepdims=True)
    base1 = jnp.sum(o1 * ao_rows, axis=1, keepdims=True)
    pos_ref[...] = jnp.zeros((8, T), jnp.int32)
    pos_ref[0:1, :] = (base0 + r0).astype(jnp.int32).reshape(1, T)
    pos_ref[1:2, :] = (base1 + r1).astype(jnp.int32).reshape(1, T)
    ti = jax.lax.broadcasted_iota(jnp.int32, (NT, E), 0).astype(_f32)
    te = jnp.sum((jnp.broadcast_to(ao_end, (NT, E)) <= ti).astype(jnp.int32),
                 axis=1, keepdims=True)
    te_ref[...] = te.reshape(1, NT)


def _dispatch(eid):
    return pl.pallas_call(
        _dispatch_body,
        grid=(1,),
        in_specs=[pl.BlockSpec((8, T), lambda i: (0, 0))],
        out_specs=[
            pl.BlockSpec((8, T), lambda i: (0, 0)),
            pl.BlockSpec((1, NT), lambda i: (0, 0)),
        ],
        out_shape=[
            jax.ShapeDtypeStruct((8, T), jnp.int32),
            jax.ShapeDtypeStruct((1, NT), jnp.int32),
        ],
    )(eid)


_SC_CH = 32
_SC_NSUB = 16


def _vector_mesh():
    return plsc.VectorSubcoreMesh(
        core_axis_name="core", subcore_axis_name="subcore")


def _sc_scatter(xb, posflat):
    stripe = T // _SC_NSUB

    @pl.kernel(
        out_type=jax.ShapeDtypeStruct((PAD_P, D), _f32),
        mesh=_vector_mesh(),
        scratch_types=[
            pltpu.VMEM((_SC_CH,), jnp.int32),
            pltpu.VMEM((_SC_CH, D), _f32),
            pltpu.SemaphoreType.DMA,
        ])
    def run(x_hbm, i_hbm, o_hbm, idx_v, rows_v, sem):
        k = jax.lax.axis_index("core")
        s = jax.lax.axis_index("subcore")
        tok_base = s * stripe

        @pl.loop(0, stripe, step=_SC_CH)
        def _(ci):
            tok0 = tok_base + ci
            pltpu.sync_copy(i_hbm.at[pl.ds(k * T + tok0, _SC_CH)], idx_v)
            pltpu.sync_copy(x_hbm.at[pl.ds(tok0, _SC_CH)], rows_v)
            pltpu.async_copy(rows_v, o_hbm.at[idx_v], sem).wait()

    return run(xb, posflat)


def _sc_gather(y, posflat):
    stripe = T // _SC_NSUB

    @pl.kernel(
        out_type=jax.ShapeDtypeStruct((2 * T, D), _f32),
        mesh=_vector_mesh(),
        scratch_types=[
            pltpu.VMEM((_SC_CH,), jnp.int32),
            pltpu.VMEM((_SC_CH, D), _f32),
            pltpu.SemaphoreType.DMA,
        ])
    def run(y_hbm, i_hbm, o_hbm, idx_v, rows_v, sem):
        k = jax.lax.axis_index("core")
        s = jax.lax.axis_index("subcore")
        row_base = k * T + s * stripe

        @pl.loop(0, stripe, step=_SC_CH)
        def _(ci):
            r0 = row_base + ci
            pltpu.sync_copy(i_hbm.at[pl.ds(r0, _SC_CH)], idx_v)
            pltpu.async_copy(y_hbm.at[idx_v], rows_v, sem).wait()
            pltpu.sync_copy(rows_v, o_hbm.at[pl.ds(r0, _SC_CH)])

    return run(y, posflat)


def _gateup_body(te_ref, xg_ref, wg_ref, wu_ref, h_ref, wgb_ref, wub_ref):
    t = pl.program_id(1)
    te = te_ref[t]
    changed = (t == 0) | (te != te_ref[jnp.maximum(t - 1, 0)])

    @pl.when(changed)
    def _():
        wgb_ref[...] = wg_ref[0].astype(_bf16)
        wub_ref[...] = wu_ref[0].astype(_bf16)

    @pl.when(te < E)
    def _():
        xb = xg_ref[...].astype(_bf16)
        g = _dot_t(xb, wgb_ref[...])
        u = _dot_t(xb, wub_ref[...])
        h_ref[0] = (_silu(g) * u).astype(_bf16)


def _gateup(te, xg, w_gate_up):
    grid_spec = pltpu.PrefetchScalarGridSpec(
        num_scalar_prefetch=1,
        grid=(NF, NT),
        in_specs=[
            pl.BlockSpec((BT, D), lambda f, t, te: (t, 0)),
            pl.BlockSpec((1, BF, D),
                         lambda f, t, te: (jnp.minimum(te[t], E - 1), f, 0)),
            pl.BlockSpec((1, BF, D),
                         lambda f, t, te: (jnp.minimum(te[t], E - 1),
                                           NF + f, 0)),
        ],
        out_specs=pl.BlockSpec((1, BT, BF), lambda f, t, te: (f, t, 0)),
        scratch_shapes=[pltpu.VMEM((BF, D), _bf16),
                        pltpu.VMEM((BF, D), _bf16)],
    )
    return pl.pallas_call(
        _gateup_body,
        grid_spec=grid_spec,
        out_shape=jax.ShapeDtypeStruct((NF, PAD_P, BF), _bf16),
    )(te, xg, w_gate_up, w_gate_up)


def _down_body(te_ref, h_ref, wd_ref, y_ref, wdb_ref):
    t = pl.program_id(0)
    te = te_ref[t]
    changed = (t == 0) | (te != te_ref[jnp.maximum(t - 1, 0)])

    @pl.when(changed)
    def _():
        wdb_ref[...] = wd_ref[0].astype(_bf16)

    @pl.when(te < E)
    def _():
        acc = _dot_t(h_ref[0], wdb_ref[:, 0:BF])
        for c in range(1, NF):
            acc = acc + _dot_t(h_ref[c], wdb_ref[:, c * BF:(c + 1) * BF])
        y_ref[...] = acc


def _down(te, h, w_down):
    grid_spec = pltpu.PrefetchScalarGridSpec(
        num_scalar_prefetch=1,
        grid=(NT,),
        in_specs=[
            pl.BlockSpec((NF, BT, BF), lambda t, te: (0, t, 0)),
            pl.BlockSpec((1, D, F),
                         lambda t, te: (jnp.minimum(te[t], E - 1), 0, 0)),
        ],
        out_specs=pl.BlockSpec((BT, D), lambda t, te: (t, 0)),
        scratch_shapes=[pltpu.VMEM((D, F), _bf16)],
    )
    return pl.pallas_call(
        _down_body,
        grid_spec=grid_spec,
        out_shape=jax.ShapeDtypeStruct((PAD_P, D), _f32),
    )(te, h, w_down)


def _shared_gateup_body(xb_ref, wg_ref, wu_ref, sh_ref, wgb_ref, wub_ref):
    t = pl.program_id(1)

    @pl.when(t == 0)
    def _():
        wgb_ref[...] = wg_ref[...].astype(_bf16)
        wub_ref[...] = wu_ref[...].astype(_bf16)

    xb = xb_ref[...]
    g = _dot_t(xb, wgb_ref[...])
    u = _dot_t(xb, wub_ref[...])
    sh_ref[0] = (_silu(g) * u).astype(_bf16)


def _shared_gateup(xb, w_shared_gate_up):
    return pl.pallas_call(
        _shared_gateup_body,
        grid=(NSF, T // BT2),
        in_specs=[
            pl.BlockSpec((BT2, D), lambda s, t: (t, 0)),
            pl.BlockSpec((BSF, D), lambda s, t: (s, 0)),
            pl.BlockSpec((BSF, D), lambda s, t: (NSF + s, 0)),
        ],
        out_specs=pl.BlockSpec((1, BT2, BSF), lambda s, t: (s, t, 0)),
        out_shape=jax.ShapeDtypeStruct((NSF, T, BSF), _bf16),
        scratch_shapes=[pltpu.VMEM((BSF, D), _bf16),
                        pltpu.VMEM((BSF, D), _bf16)],
    )(xb, w_shared_gate_up, w_shared_gate_up)


def _shared_down_body(sh_ref, wd_ref, y0_ref, y1_ref, tw_ref, out_ref,
                      wdb_ref):
    t = pl.program_id(1)

    @pl.when(t == 0)
    def _():
        wdb_ref[...] = wd_ref[...].astype(_bf16)

    acc = _dot_t(sh_ref[0], wdb_ref[:, 0:BSF])
    for c in range(1, NSF):
        acc = acc + _dot_t(sh_ref[c], wdb_ref[:, c * BSF:(c + 1) * BSF])
    tw0 = tw_ref[0:1, :].reshape(BT2, 1)
    tw1 = tw_ref[1:2, :].reshape(BT2, 1)
    out_ref[...] = acc + tw0 * y0_ref[...] + tw1 * y1_ref[...]


def _shared_down_combine(sh, w_shared_down, yg, tw):
    nt2 = T // BT2
    return pl.pallas_call(
        _shared_down_body,
        grid=(D // BD2, nt2),
        in_specs=[
            pl.BlockSpec((NSF, BT2, BSF), lambda d, t: (0, t, 0)),
            pl.BlockSpec((BD2, SF), lambda d, t: (d, 0),
                         pipeline_mode=pl.Buffered(buffer_count=1)),
            pl.BlockSpec((BT2, BD2), lambda d, t: (t, d)),
            pl.BlockSpec((BT2, BD2), lambda d, t: (nt2 + t, d)),
            pl.BlockSpec((8, BT2), lambda d, t: (0, t)),
        ],
        out_specs=pl.BlockSpec((BT2, BD2), lambda d, t: (t, d)),
        out_shape=jax.ShapeDtypeStruct((T, D), _f32),
        scratch_shapes=[pltpu.VMEM((BD2, SF), _bf16)],
    )(sh, w_shared_down, yg, yg, tw)


def kernel(x, w_gate, w_gate_up, w_down, w_shared_gate_up, w_shared_down):
    xb = x.astype(_bf16)
    eid, tw = _router(x, w_gate)
    pos8, te2d = _dispatch(eid)
    posflat = pos8[:2].reshape(2 * T)
    te = te2d.reshape(NT)
    xg = _sc_scatter(x, posflat)
    h = _gateup(te, xg, w_gate_up)
    y = _down(te, h, w_down)
    yg = _sc_gather(y, posflat)
    sh = _shared_gateup(xb, w_shared_gate_up)
    return _shared_down_combine(sh, w_shared_down, yg, tw)
